# baseline (speedup 1.0000x reference)
import functools

import jax
import jax.numpy as jnp
from jax import lax
from jax.experimental import pallas as pl
from jax.experimental.pallas import tpu as pltpu
from jax.experimental.pallas import tpu_sc as plsc

_N = 10000
_E = 320000
_D = 128
_H = 64
_G = 256

_NC = 2
_NS = 16
_NW = _NC * _NS
_CHUNK = 128
_NCHT = _E // _CHUNK
_NCHUNK = _NCHT // _NW
_NTAIL = _NCHT - _NCHUNK * _NW
_RPT = _N // _NS

_BLK = 1000
_NBLK = _N // _BLK


_K = 3
_NGRP = _NCHUNK // _K


def _seg_sum_sc_body(feat, src2, dst2, zinit, out, src_v, dst_v,
                     src_t, dst_t, rows, acc, gsem, ssem):
    c = lax.axis_index("c")
    s = lax.axis_index("s")
    wid = c * _NS + s

    pltpu.sync_copy(zinit, acc.at[pl.ds(s * _RPT, _RPT)])
    pltpu.sync_copy(src2.at[pl.ds(wid * _NCHUNK, _NCHUNK)], src_v)
    pltpu.sync_copy(dst2.at[pl.ds(wid * _NCHUNK, _NCHUNK)], dst_v)

    @pl.when(wid < _NTAIL)
    def _():
        pltpu.sync_copy(src2.at[pl.ds(_NCHUNK * _NW + wid, 1)], src_t)
        pltpu.sync_copy(dst2.at[pl.ds(_NCHUNK * _NW + wid, 1)], dst_t)

    plsc.subcore_barrier()

    def gather_wait(slot):
        pltpu.make_async_copy(feat.at[src_v.at[0]], rows.at[slot],
                              gsem.at[slot]).wait()

    def scatter_wait(slot):
        pltpu.make_async_copy(rows.at[slot], acc.at[dst_v.at[0]],
                              ssem.at[slot]).wait()

    for b in range(_K):
        pltpu.async_copy(feat.at[src_v.at[b]], rows.at[b], gsem.at[b])

    def outer(j, carry):
        for h in (0, 1):
            g = 2 * j + h

            @pl.when(g < _NGRP)
            def _():
                for b in range(_K):
                    slot = h * _K + b
                    gather_wait(slot)
                    pltpu.async_copy(rows.at[slot],
                                     acc.at[dst_v.at[g * _K + b]],
                                     ssem.at[slot], add=True)

                @pl.when(g + 1 < _NGRP)
                def _():
                    for b in range(_K):
                        slot = (1 - h) * _K + b

                        @pl.when(g >= 1)
                        def _():
                            scatter_wait(slot)

                        pltpu.async_copy(
                            feat.at[src_v.at[(g + 1) * _K + b]],
                            rows.at[slot], gsem.at[slot])
        return carry

    lax.fori_loop(0, (_NGRP + 2) // 2, outer, 0)
    for b in range(2 * _K):
        scatter_wait(b)

    @pl.when(wid < _NTAIL)
    def _():
        pltpu.async_copy(feat.at[src_t.at[0]], rows.at[0], gsem.at[0]).wait()
        pltpu.async_copy(rows.at[0], acc.at[dst_t.at[0]], ssem.at[0],
                         add=True).wait()

    plsc.subcore_barrier()
    pltpu.sync_copy(acc.at[pl.ds(s * _RPT, _RPT)],
                    out.at[c].at[pl.ds(s * _RPT, _RPT)])


_seg_sum_sc = pl.kernel(
    _seg_sum_sc_body,
    out_type=jax.ShapeDtypeStruct((_NC, _N, _H), jnp.float32),
    mesh=plsc.VectorSubcoreMesh(core_axis_name="c", subcore_axis_name="s"),
    scratch_types=[
        pltpu.VMEM((_NCHUNK, _CHUNK), jnp.int32),
        pltpu.VMEM((_NCHUNK, _CHUNK), jnp.int32),
        pltpu.VMEM((1, _CHUNK), jnp.int32),
        pltpu.VMEM((1, _CHUNK), jnp.int32),
        pltpu.VMEM((2 * _K, _CHUNK, _H), jnp.float32),
        pltpu.VMEM_SHARED((_N, _H), jnp.float32),
        pltpu.SemaphoreType.DMA((2 * _K,)),
        pltpu.SemaphoreType.DMA((2 * _K,)),
    ],
    compiler_params=pltpu.CompilerParams(use_tc_tiling_on_sc=False),
)


def _mm_body(x_ref, w_ref, o_ref):
    o_ref[...] = jax.lax.dot_general(
        x_ref[...], w_ref[...], (((1,), (0,)), ((), ())),
        preferred_element_type=jnp.float32)


def _proj(x, w):
    k = x.shape[1]
    return pl.pallas_call(
        _mm_body,
        grid=(_NBLK,),
        in_specs=[
            pl.BlockSpec((_BLK, k), lambda i: (i, 0)),
            pl.BlockSpec((k, _H), lambda i: (0, 0)),
        ],
        out_specs=pl.BlockSpec((_BLK, _H), lambda i: (i, 0)),
        out_shape=jax.ShapeDtypeStruct((_N, _H), jnp.float32),
    )(x, w)


def _mid_body(xa_ref, p_ref, b1a_ref, w1b_ref, b1b_ref, w2a_ref, o_ref):
    z = xa_ref[...] + p_ref[0] + p_ref[1] + b1a_ref[...]
    z = jnp.maximum(z, 0.0)
    h = jax.lax.dot_general(z, w1b_ref[...], (((1,), (0,)), ((), ())),
                            preferred_element_type=jnp.float32)
    h = jnp.maximum(h + b1b_ref[...], 0.0)
    o_ref[...] = jax.lax.dot_general(h, w2a_ref[...], (((1,), (0,)), ((), ())),
                                     preferred_element_type=jnp.float32)


def _mid(xa, parts, b1a, w1b, b1b, w2a):
    return pl.pallas_call(
        _mid_body,
        grid=(_NBLK,),
        in_specs=[
            pl.BlockSpec((_BLK, _H), lambda i: (i, 0)),
            pl.BlockSpec((_NC, _BLK, _H), lambda i: (0, i, 0)),
            pl.BlockSpec((1, _H), lambda i: (0, 0)),
            pl.BlockSpec((_H, _H), lambda i: (0, 0)),
            pl.BlockSpec((1, _H), lambda i: (0, 0)),
            pl.BlockSpec((_H, _H), lambda i: (0, 0)),
        ],
        out_specs=pl.BlockSpec((_BLK, _H), lambda i: (i, 0)),
        out_shape=jax.ShapeDtypeStruct((_N, _H), jnp.float32),
    )(xa, parts, b1a, w1b, b1b, w2a)


def _pool_body(ha_ref, q_ref, b2a_ref, w2b_ref, b2b_ref, batch_ref,
               we_ref, be_ref, wp_ref, bp_ref,
               hg_ref, e_ref, p_ref, acc_ref, cnt_ref):
    i = pl.program_id(0)

    z = ha_ref[...] + q_ref[0] + q_ref[1] + b2a_ref[...]
    z = jnp.maximum(z, 0.0)
    h2 = jax.lax.dot_general(z, w2b_ref[...], (((1,), (0,)), ((), ())),
                             preferred_element_type=jnp.float32)
    h2 = jnp.maximum(h2 + b2b_ref[...], 0.0)

    gids = jax.lax.broadcasted_iota(jnp.int32, (1, _G), 1)
    m = (batch_ref[...] == gids).astype(jnp.float32)

    @pl.when(i == 0)
    def _():
        acc_ref[...] = jnp.zeros_like(acc_ref)
        cnt_ref[...] = jnp.zeros_like(cnt_ref)

    acc_ref[...] += jax.lax.dot_general(
        m, h2, (((0,), (0,)), ((), ())), preferred_element_type=jnp.float32)
    cnt_ref[...] += jax.lax.dot_general(
        m, jnp.ones((_BLK, 1), jnp.float32), (((0,), (0,)), ((), ())),
        preferred_element_type=jnp.float32)

    @pl.when(i == _NBLK - 1)
    def _():
        hg = acc_ref[...] / jnp.maximum(cnt_ref[...], 1.0)
        hg_ref[...] = hg
        e_ref[...] = jax.lax.dot_general(
            hg, we_ref[...], (((1,), (0,)), ((), ())),
            preferred_element_type=jnp.float32) + be_ref[...]
        p_ref[...] = jax.lax.dot_general(
            hg, wp_ref[...], (((1,), (0,)), ((), ())),
            preferred_element_type=jnp.float32) + bp_ref[...]


def _pool(ha, parts, b2a, w2b, b2b, batch2, we, be, wp, bp):
    return pl.pallas_call(
        _pool_body,
        grid=(_NBLK,),
        in_specs=[
            pl.BlockSpec((_BLK, _H), lambda i: (i, 0)),
            pl.BlockSpec((_NC, _BLK, _H), lambda i: (0, i, 0)),
            pl.BlockSpec((1, _H), lambda i: (0, 0)),
            pl.BlockSpec((_H, _H), lambda i: (0, 0)),
            pl.BlockSpec((1, _H), lambda i: (0, 0)),
            pl.BlockSpec((_BLK, 1), lambda i: (i, 0)),
            pl.BlockSpec((_H, 1), lambda i: (0, 0)),
            pl.BlockSpec((1, 1), lambda i: (0, 0)),
            pl.BlockSpec((_H, 6), lambda i: (0, 0)),
            pl.BlockSpec((1, 6), lambda i: (0, 0)),
        ],
        out_specs=[
            pl.BlockSpec((_G, _H), lambda i: (0, 0)),
            pl.BlockSpec((_G, 1), lambda i: (0, 0)),
            pl.BlockSpec((_G, 6), lambda i: (0, 0)),
        ],
        out_shape=[
            jax.ShapeDtypeStruct((_G, _H), jnp.float32),
            jax.ShapeDtypeStruct((_G, 1), jnp.float32),
            jax.ShapeDtypeStruct((_G, 6), jnp.float32),
        ],
        scratch_shapes=[
            pltpu.VMEM((_G, _H), jnp.float32),
            pltpu.VMEM((_G, 1), jnp.float32),
        ],
    )(ha, parts, b2a, w2b, b2b, batch2, we, be, wp, bp)


@jax.jit
def kernel(x, edge_index, batch, W1a, b1a, W1b, b1b, W2a, b2a, W2b, b2b,
           We, be, Wp, bp):
    src2 = edge_index[0].reshape(_NCHT, _CHUNK)
    dst2 = edge_index[1].reshape(_NCHT, _CHUNK)
    zinit = jnp.zeros((_RPT, _H), jnp.float32)
    batch2 = batch.reshape(_N, 1)

    xa = _proj(x, W1a)
    p1 = _seg_sum_sc(xa, src2, dst2, zinit)
    h1a = _mid(xa, p1, b1a.reshape(1, _H), W1b, b1b.reshape(1, _H), W2a)
    p2 = _seg_sum_sc(h1a, src2, dst2, zinit)
    hg, e, p = _pool(h1a, p2, b2a.reshape(1, _H), W2b, b2b.reshape(1, _H),
                     batch2, We, be.reshape(1, 1), Wp, bp.reshape(1, 6))
    return hg, e, p


# speedup vs baseline: 1.0226x; 1.0226x over previous
import jax
import jax.numpy as jnp
from jax import lax
from jax.experimental import pallas as pl
from jax.experimental.pallas import tpu as pltpu
from jax.experimental.pallas import tpu_sc as plsc

_N = 10000
_E = 320000
_D = 128
_H = 64
_G = 256

_NC = 2
_NS = 16
_NW = _NC * _NS
_EPW = _E // _NW
_CHUNK = 80
_NCHUNK = _EPW // _CHUNK
_NACC = 10048
_IPT = _NACC // _NS
_OPT = _N // _NS

_BLK = 1000
_NBLK = _N // _BLK


_K = 5
_NGRP = _NCHUNK // _K


def _seg_sum_sc_body(feat, src2, dst2, zinit, out, src_v, dst_v, rows, acc,
                     gsem, ssem):
    c = lax.axis_index("c")
    s = lax.axis_index("s")
    wid = c * _NS + s

    pltpu.sync_copy(zinit, acc.at[pl.ds(s * _IPT, _IPT)])
    pltpu.sync_copy(src2.at[wid], src_v)
    pltpu.sync_copy(dst2.at[wid], dst_v)
    plsc.subcore_barrier()

    def gather_wait(slot):
        pltpu.make_async_copy(feat.at[src_v.at[0]], rows.at[slot],
                              gsem.at[slot]).wait()

    def scatter_wait(slot):
        pltpu.make_async_copy(rows.at[slot], acc.at[dst_v.at[0]],
                              ssem.at[slot]).wait()

    for b in range(_K):
        pltpu.async_copy(feat.at[src_v.at[b]], rows.at[b], gsem.at[b])

    def outer(j, carry):
        for h in (0, 1):
            g = 2 * j + h

            @pl.when(g < _NGRP)
            def _():
                for b in range(_K):
                    slot = h * _K + b
                    gather_wait(slot)
                    pltpu.async_copy(rows.at[slot],
                                     acc.at[dst_v.at[g * _K + b]],
                                     ssem.at[slot], add=True)

                @pl.when(g + 1 < _NGRP)
                def _():
                    for b in range(_K):
                        slot = (1 - h) * _K + b

                        @pl.when(g >= 1)
                        def _():
                            scatter_wait(slot)

                        pltpu.async_copy(
                            feat.at[src_v.at[(g + 1) * _K + b]],
                            rows.at[slot], gsem.at[slot])
        return carry

    lax.fori_loop(0, (_NGRP + 2) // 2, outer, 0)
    for b in range(2 * _K):
        scatter_wait(b)
    plsc.subcore_barrier()
    pltpu.sync_copy(acc.at[pl.ds(s * _OPT, _OPT)],
                    out.at[c].at[pl.ds(s * _OPT, _OPT)])


_seg_sum_sc = pl.kernel(
    _seg_sum_sc_body,
    out_type=jax.ShapeDtypeStruct((_NC, _N, _H), jnp.float32),
    mesh=plsc.VectorSubcoreMesh(core_axis_name="c", subcore_axis_name="s"),
    scratch_types=[
        pltpu.VMEM((_NCHUNK, _CHUNK), jnp.int32),
        pltpu.VMEM((_NCHUNK, _CHUNK), jnp.int32),
        pltpu.VMEM((2 * _K, _CHUNK, _H), jnp.float32),
        pltpu.VMEM_SHARED((_NACC, _H), jnp.float32),
        pltpu.SemaphoreType.DMA((2 * _K,)),
        pltpu.SemaphoreType.DMA((2 * _K,)),
    ],
    compiler_params=pltpu.CompilerParams(use_tc_tiling_on_sc=False),
)


def _mm_body(x_ref, w_ref, o_ref):
    o_ref[...] = jax.lax.dot_general(
        x_ref[...], w_ref[...], (((1,), (0,)), ((), ())),
        preferred_element_type=jnp.float32)


def _proj(x, w):
    k = x.shape[1]
    return pl.pallas_call(
        _mm_body,
        grid=(_NBLK,),
        in_specs=[
            pl.BlockSpec((_BLK, k), lambda i: (i, 0)),
            pl.BlockSpec((k, _H), lambda i: (0, 0)),
        ],
        out_specs=pl.BlockSpec((_BLK, _H), lambda i: (i, 0)),
        out_shape=jax.ShapeDtypeStruct((_N, _H), jnp.float32),
    )(x, w)


def _mid_body(xa_ref, p_ref, b1a_ref, w1b_ref, b1b_ref, w2a_ref, o_ref):
    z = xa_ref[...] + p_ref[0] + p_ref[1] + b1a_ref[...]
    z = jnp.maximum(z, 0.0)
    h = jax.lax.dot_general(z, w1b_ref[...], (((1,), (0,)), ((), ())),
                            preferred_element_type=jnp.float32)
    h = jnp.maximum(h + b1b_ref[...], 0.0)
    o_ref[...] = jax.lax.dot_general(h, w2a_ref[...], (((1,), (0,)), ((), ())),
                                     preferred_element_type=jnp.float32)


def _mid(xa, parts, b1a, w1b, b1b, w2a):
    return pl.pallas_call(
        _mid_body,
        grid=(_NBLK,),
        in_specs=[
            pl.BlockSpec((_BLK, _H), lambda i: (i, 0)),
            pl.BlockSpec((_NC, _BLK, _H), lambda i: (0, i, 0)),
            pl.BlockSpec((1, _H), lambda i: (0, 0)),
            pl.BlockSpec((_H, _H), lambda i: (0, 0)),
            pl.BlockSpec((1, _H), lambda i: (0, 0)),
            pl.BlockSpec((_H, _H), lambda i: (0, 0)),
        ],
        out_specs=pl.BlockSpec((_BLK, _H), lambda i: (i, 0)),
        out_shape=jax.ShapeDtypeStruct((_N, _H), jnp.float32),
    )(xa, parts, b1a, w1b, b1b, w2a)


def _pool_body(ha_ref, q_ref, b2a_ref, w2b_ref, b2b_ref, batch_ref,
               we_ref, be_ref, wp_ref, bp_ref,
               hg_ref, e_ref, p_ref, acc_ref, cnt_ref):
    i = pl.program_id(0)

    z = ha_ref[...] + q_ref[0] + q_ref[1] + b2a_ref[...]
    z = jnp.maximum(z, 0.0)
    h2 = jax.lax.dot_general(z, w2b_ref[...], (((1,), (0,)), ((), ())),
                             preferred_element_type=jnp.float32)
    h2 = jnp.maximum(h2 + b2b_ref[...], 0.0)

    gids = jax.lax.broadcasted_iota(jnp.int32, (1, _G), 1)
    m = (batch_ref[...] == gids).astype(jnp.float32)

    @pl.when(i == 0)
    def _():
        acc_ref[...] = jnp.zeros_like(acc_ref)
        cnt_ref[...] = jnp.zeros_like(cnt_ref)

    acc_ref[...] += jax.lax.dot_general(
        m, h2, (((0,), (0,)), ((), ())), preferred_element_type=jnp.float32)
    cnt_ref[...] += jax.lax.dot_general(
        m, jnp.ones((_BLK, 1), jnp.float32), (((0,), (0,)), ((), ())),
        preferred_element_type=jnp.float32)

    @pl.when(i == _NBLK - 1)
    def _():
        hg = acc_ref[...] / jnp.maximum(cnt_ref[...], 1.0)
        hg_ref[...] = hg
        e_ref[...] = jax.lax.dot_general(
            hg, we_ref[...], (((1,), (0,)), ((), ())),
            preferred_element_type=jnp.float32) + be_ref[...]
        p_ref[...] = jax.lax.dot_general(
            hg, wp_ref[...], (((1,), (0,)), ((), ())),
            preferred_element_type=jnp.float32) + bp_ref[...]


def _pool(ha, parts, b2a, w2b, b2b, batch2, we, be, wp, bp):
    return pl.pallas_call(
        _pool_body,
        grid=(_NBLK,),
        in_specs=[
            pl.BlockSpec((_BLK, _H), lambda i: (i, 0)),
            pl.BlockSpec((_NC, _BLK, _H), lambda i: (0, i, 0)),
            pl.BlockSpec((1, _H), lambda i: (0, 0)),
            pl.BlockSpec((_H, _H), lambda i: (0, 0)),
            pl.BlockSpec((1, _H), lambda i: (0, 0)),
            pl.BlockSpec((_BLK, 1), lambda i: (i, 0)),
            pl.BlockSpec((_H, 1), lambda i: (0, 0)),
            pl.BlockSpec((1, 1), lambda i: (0, 0)),
            pl.BlockSpec((_H, 6), lambda i: (0, 0)),
            pl.BlockSpec((1, 6), lambda i: (0, 0)),
        ],
        out_specs=[
            pl.BlockSpec((_G, _H), lambda i: (0, 0)),
            pl.BlockSpec((_G, 1), lambda i: (0, 0)),
            pl.BlockSpec((_G, 6), lambda i: (0, 0)),
        ],
        out_shape=[
            jax.ShapeDtypeStruct((_G, _H), jnp.float32),
            jax.ShapeDtypeStruct((_G, 1), jnp.float32),
            jax.ShapeDtypeStruct((_G, 6), jnp.float32),
        ],
        scratch_shapes=[
            pltpu.VMEM((_G, _H), jnp.float32),
            pltpu.VMEM((_G, 1), jnp.float32),
        ],
    )(ha, parts, b2a, w2b, b2b, batch2, we, be, wp, bp)


@jax.jit
def kernel(x, edge_index, batch, W1a, b1a, W1b, b1b, W2a, b2a, W2b, b2b,
           We, be, Wp, bp):
    src2 = edge_index[0].reshape(_NW, _NCHUNK, _CHUNK)
    dst2 = edge_index[1].reshape(_NW, _NCHUNK, _CHUNK)
    zinit = jnp.zeros((_IPT, _H), jnp.float32)
    batch2 = batch.reshape(_N, 1)

    xa = _proj(x, W1a)
    p1 = _seg_sum_sc(xa, src2, dst2, zinit)
    h1a = _mid(xa, p1, b1a.reshape(1, _H), W1b, b1b.reshape(1, _H), W2a)
    p2 = _seg_sum_sc(h1a, src2, dst2, zinit)
    hg, e, p = _pool(h1a, p2, b2a.reshape(1, _H), W2b, b2b.reshape(1, _H),
                     batch2, We, be.reshape(1, 1), Wp, bp.reshape(1, 6))
    return hg, e, p
